# baseline (speedup 1.0000x reference)
import functools

import jax
import jax.numpy as jnp
from jax import lax
from jax.experimental import pallas as pl
from jax.experimental.pallas import tpu as pltpu
from jax.experimental.pallas import tpu_sc as plsc

_N, _K = 16384, 26
_NW = 32
_RPT = _N // _NW
_CR = 128
_NCH = _RPT // _CR
_RU = 4

_LN2 = 0.6931471805599453
_C0 = 0.0001415121753789439 - 127.0 * _LN2
_C1 = 0.9954273382579881
_C2 = -0.4640725804471214
_C3 = 0.21641043832781495
_C4 = -0.05486285286206372


def _log_term(x):
    x = jnp.minimum(jnp.maximum(x, 0.01), 0.99)
    bits = lax.bitcast_convert_type(x, jnp.int32)
    eb = bits >> 23
    m = (bits & 0x7FFFFF) | 0x3F800000
    u = lax.bitcast_convert_type(m, jnp.float32) - 1.0
    r = _C4
    r = r * u + _C3
    r = r * u + _C2
    r = r * u + _C1
    r = r * u + _C0
    return eb.astype(jnp.float32) * _LN2 + r


_mesh = plsc.VectorSubcoreMesh(core_axis_name="c", subcore_axis_name="s")


@functools.partial(
    pl.kernel,
    mesh=_mesh,
    compiler_params=pltpu.CompilerParams(use_tc_tiling_on_sc=True),
    out_type=jax.ShapeDtypeStruct((2, _NW, 16), jnp.float32),
    scratch_types=[
        pltpu.VMEM((_CR, _K), jnp.float32),
        pltpu.VMEM((_CR, _K), jnp.float32),
        pltpu.VMEM((_CR, _K), jnp.float32),
        pltpu.VMEM((_CR, _K), jnp.float32),
        pltpu.VMEM((16,), jnp.float32),
        pltpu.SemaphoreType.DMA,
        pltpu.SemaphoreType.DMA,
    ],
)
def _sc_loss(p1, p2, q1, q2, out, a0, b0, a1, b1, acc_v, s0, s1):
    wid = lax.axis_index("s") * 2 + lax.axis_index("c")
    row0 = wid * _RPT
    bufs = ((a0, b0), (a1, b1))
    sems = (s0, s1)
    chunks = []
    for pr, po in ((p1, q2), (p2, q1)):
        for c in range(_NCH):
            chunks.append((pr, po, c * _CR))

    def start(idx):
        pr, po, roff = chunks[idx]
        par = idx % 2
        return (
            pltpu.async_copy(pr.at[pl.ds(row0 + roff, _CR), :], bufs[par][0], sems[par]),
            pltpu.async_copy(po.at[pl.ds(row0 + roff, _CR), :], bufs[par][1], sems[par]),
        )

    zero = jnp.zeros((16,), jnp.float32)
    tailmask = lax.iota(jnp.int32, 16) >= 6

    def make_body(pr_v, po_v):
        def body(j, accs):
            r0 = j * _RU
            new = []
            for r in range(_RU):
                x0 = pr_v[r0 + r, pl.ds(0, 16)]
                w0 = po_v[r0 + r, pl.ds(0, 16)]
                x1 = pr_v[r0 + r, pl.ds(10, 16)]
                w1 = po_v[r0 + r, pl.ds(10, 16)]
                t0 = w0 * _log_term(x0)
                t1 = jnp.where(tailmask, w1 * _log_term(x1), 0.0)
                new.append(accs[2 * r] + t0)
                new.append(accs[2 * r + 1] + t1)
            return tuple(new)
        return body

    nacc = 2 * _RU
    bacc = [zero, zero]
    cps = start(0)
    for idx in range(2 * _NCH):
        nxt = start(idx + 1) if idx + 1 < 2 * _NCH else None
        cps[0].wait()
        cps[1].wait()
        par = idx % 2
        accs = lax.fori_loop(0, _CR // _RU,
                             make_body(bufs[par][0], bufs[par][1]),
                             (zero,) * nacc)
        tot = ((accs[0] + accs[1]) + (accs[2] + accs[3])) + \
              ((accs[4] + accs[5]) + (accs[6] + accs[7]))
        b = idx // _NCH
        bacc[b] = bacc[b] + tot
        cps = nxt

    acc_v[...] = bacc[0]
    pltpu.sync_copy(acc_v, out.at[0, wid])
    acc_v[...] = bacc[1]
    pltpu.sync_copy(acc_v, out.at[1, wid])


def kernel(prior_1, prior_2, post_1, post_2):
    parts = _sc_loss(prior_1, prior_2, post_1, post_2)
    losses = 0.05 - jnp.sum(parts, axis=(1, 2)) / _N
    return (losses[0], losses[1])


# speedup vs baseline: 1.6859x; 1.6859x over previous
import functools

import jax
import jax.numpy as jnp
from jax import lax
from jax.experimental import pallas as pl
from jax.experimental.pallas import tpu as pltpu
from jax.experimental.pallas import tpu_sc as plsc

_N, _K = 16384, 26
_NW = 32
_CPT = _N // _NW
_NT = _CPT // 16

_LN2 = 0.6931471805599453
_C0 = 0.0001415121753789439 - 127.0 * _LN2
_C1 = 0.9954273382579881
_C2 = -0.4640725804471214
_C3 = 0.21641043832781495
_C4 = -0.05486285286206372


def _log_term(x):
    x = jnp.minimum(jnp.maximum(x, 0.01), 0.99)
    bits = lax.bitcast_convert_type(x, jnp.int32)
    eb = bits >> 23
    m = (bits & 0x7FFFFF) | 0x3F800000
    u = lax.bitcast_convert_type(m, jnp.float32) - 1.0
    r = _C4
    r = r * u + _C3
    r = r * u + _C2
    r = r * u + _C1
    r = r * u + _C0
    return eb.astype(jnp.float32) * _LN2 + r


_mesh = plsc.VectorSubcoreMesh(core_axis_name="c", subcore_axis_name="s")


@functools.partial(
    pl.kernel,
    mesh=_mesh,
    out_type=jax.ShapeDtypeStruct((2, _NW, 16), jnp.float32),
    scratch_types=[
        pltpu.VMEM((_K, _CPT), jnp.float32),
        pltpu.VMEM((_K, _CPT), jnp.float32),
        pltpu.VMEM((_K, _CPT), jnp.float32),
        pltpu.VMEM((_K, _CPT), jnp.float32),
        pltpu.VMEM((16,), jnp.float32),
        pltpu.SemaphoreType.DMA,
        pltpu.SemaphoreType.DMA,
        pltpu.SemaphoreType.DMA,
        pltpu.SemaphoreType.DMA,
    ],
)
def _sc_loss(p1, p2, q1, q2, out, a_v, b_v, c_v, d_v, acc_v,
             s1, s2, s3, s4):
    wid = lax.axis_index("s") * 2 + lax.axis_index("c")
    col0 = wid * _CPT
    cp1 = pltpu.async_copy(p1.at[:, pl.ds(col0, _CPT)], a_v, s1)
    cp2 = pltpu.async_copy(q2.at[:, pl.ds(col0, _CPT)], b_v, s2)
    cp3 = pltpu.async_copy(p2.at[:, pl.ds(col0, _CPT)], c_v, s3)
    cp4 = pltpu.async_copy(q1.at[:, pl.ds(col0, _CPT)], d_v, s4)

    zero = jnp.zeros((16,), jnp.float32)
    nacc = 8

    def make_body(pr_v, po_v):
        def body(j, accs):
            accs = list(accs)
            c = j * 16
            for r in range(_K):
                x = pr_v[r, pl.ds(c, 16)]
                w = po_v[r, pl.ds(c, 16)]
                accs[r % nacc] = accs[r % nacc] + w * _log_term(x)
            return tuple(accs)
        return body

    cp1.wait()
    cp2.wait()
    accs = lax.fori_loop(0, _NT, make_body(a_v, b_v), (zero,) * nacc)
    acc1 = ((accs[0] + accs[1]) + (accs[2] + accs[3])) + \
           ((accs[4] + accs[5]) + (accs[6] + accs[7]))

    cp3.wait()
    cp4.wait()
    accs = lax.fori_loop(0, _NT, make_body(c_v, d_v), (zero,) * nacc)
    acc2 = ((accs[0] + accs[1]) + (accs[2] + accs[3])) + \
           ((accs[4] + accs[5]) + (accs[6] + accs[7]))

    acc_v[...] = acc1
    pltpu.sync_copy(acc_v, out.at[0, wid])
    acc_v[...] = acc2
    pltpu.sync_copy(acc_v, out.at[1, wid])


def kernel(prior_1, prior_2, post_1, post_2):
    parts = _sc_loss(prior_1.T, prior_2.T, post_1.T, post_2.T)
    losses = 0.05 - jnp.sum(parts, axis=(1, 2)) / _N
    return (losses[0], losses[1])


# speedup vs baseline: 1.6886x; 1.0016x over previous
import functools

import jax
import jax.numpy as jnp
from jax import lax
from jax.experimental import pallas as pl
from jax.experimental.pallas import tpu as pltpu
from jax.experimental.pallas import tpu_sc as plsc

_N, _K = 16384, 26
_NW = 32
_CPT = _N // _NW
_NT = _CPT // 16

_LN2 = 0.6931471805599453
_C0 = 0.0001415121753789439 - 127.0 * _LN2
_C1 = 0.9954273382579881
_C2 = -0.4640725804471214
_C3 = 0.21641043832781495
_C4 = -0.05486285286206372


def _log_term(x):
    x = jnp.minimum(jnp.maximum(x, 0.01), 0.99)
    bits = lax.bitcast_convert_type(x, jnp.int32)
    eb = bits >> 23
    m = (bits & 0x7FFFFF) | 0x3F800000
    u = lax.bitcast_convert_type(m, jnp.float32) - 1.0
    r = _C4
    r = r * u + _C3
    r = r * u + _C2
    r = r * u + _C1
    r = r * u + _C0
    return eb.astype(jnp.float32) * _LN2 + r


_mesh = plsc.VectorSubcoreMesh(core_axis_name="c", subcore_axis_name="s")


@functools.partial(
    pl.kernel,
    mesh=_mesh,
    compiler_params=pltpu.CompilerParams(skip_device_barrier=True),
    out_type=jax.ShapeDtypeStruct((2, _NW, 16), jnp.float32),
    scratch_types=[
        pltpu.VMEM((_K, _CPT), jnp.float32),
        pltpu.VMEM((_K, _CPT), jnp.float32),
        pltpu.VMEM((_K, _CPT), jnp.float32),
        pltpu.VMEM((_K, _CPT), jnp.float32),
        pltpu.VMEM((16,), jnp.float32),
        pltpu.SemaphoreType.DMA,
        pltpu.SemaphoreType.DMA,
        pltpu.SemaphoreType.DMA,
        pltpu.SemaphoreType.DMA,
    ],
)
def _sc_loss(p1, p2, q1, q2, out, a_v, b_v, c_v, d_v, acc_v,
             s1, s2, s3, s4):
    wid = lax.axis_index("s") * 2 + lax.axis_index("c")
    col0 = wid * _CPT
    cp1 = pltpu.async_copy(p1.at[:, pl.ds(col0, _CPT)], a_v, s1)
    cp2 = pltpu.async_copy(q2.at[:, pl.ds(col0, _CPT)], b_v, s2)
    cp3 = pltpu.async_copy(p2.at[:, pl.ds(col0, _CPT)], c_v, s3)
    cp4 = pltpu.async_copy(q1.at[:, pl.ds(col0, _CPT)], d_v, s4)

    zero = jnp.zeros((16,), jnp.float32)
    nacc = 8

    def make_body(pr_v, po_v):
        def body(j, accs):
            accs = list(accs)
            c = j * 16
            for r in range(_K):
                x = pr_v[r, pl.ds(c, 16)]
                w = po_v[r, pl.ds(c, 16)]
                accs[r % nacc] = accs[r % nacc] + w * _log_term(x)
            return tuple(accs)
        return body

    cp1.wait()
    cp2.wait()
    accs = lax.fori_loop(0, _NT, make_body(a_v, b_v), (zero,) * nacc)
    acc1 = ((accs[0] + accs[1]) + (accs[2] + accs[3])) + \
           ((accs[4] + accs[5]) + (accs[6] + accs[7]))

    cp3.wait()
    cp4.wait()
    accs = lax.fori_loop(0, _NT, make_body(c_v, d_v), (zero,) * nacc)
    acc2 = ((accs[0] + accs[1]) + (accs[2] + accs[3])) + \
           ((accs[4] + accs[5]) + (accs[6] + accs[7]))

    acc_v[...] = acc1
    pltpu.sync_copy(acc_v, out.at[0, wid])
    acc_v[...] = acc2
    pltpu.sync_copy(acc_v, out.at[1, wid])


def kernel(prior_1, prior_2, post_1, post_2):
    parts = _sc_loss(prior_1.T, prior_2.T, post_1.T, post_2.T)
    losses = 0.05 - jnp.sum(parts, axis=(1, 2)) / _N
    return (losses[0], losses[1])


# speedup vs baseline: 1.9863x; 1.1763x over previous
import functools

import jax
import jax.numpy as jnp
from jax import lax
from jax.experimental import pallas as pl
from jax.experimental.pallas import tpu as pltpu
from jax.experimental.pallas import tpu_sc as plsc

_N, _K = 16384, 26
_NW = 32
_SC_COLS = 8192
_BC = 2048
_CPT = _SC_COLS // _NW
_NT = _CPT // 16

_LN2 = 0.6931471805599453
_C0 = 0.0001415121753789439 - 127.0 * _LN2
_C1 = 0.9954273382579881
_C2 = -0.4640725804471214
_C3 = 0.21641043832781495
_C4 = -0.05486285286206372


def _log_term(x):
    x = jnp.minimum(jnp.maximum(x, 0.01), 0.99)
    bits = lax.bitcast_convert_type(x, jnp.int32)
    eb = bits >> 23
    m = (bits & 0x7FFFFF) | 0x3F800000
    u = lax.bitcast_convert_type(m, jnp.float32) - 1.0
    r = _C4
    r = r * u + _C3
    r = r * u + _C2
    r = r * u + _C1
    r = r * u + _C0
    return eb.astype(jnp.float32) * _LN2 + r


_mesh = plsc.VectorSubcoreMesh(core_axis_name="c", subcore_axis_name="s")


@functools.partial(
    pl.kernel,
    mesh=_mesh,
    out_type=jax.ShapeDtypeStruct((2, _NW, 16), jnp.float32),
    scratch_types=[
        pltpu.VMEM((_K, _CPT), jnp.float32),
        pltpu.VMEM((_K, _CPT), jnp.float32),
        pltpu.VMEM((_K, _CPT), jnp.float32),
        pltpu.VMEM((_K, _CPT), jnp.float32),
        pltpu.VMEM((16,), jnp.float32),
        pltpu.SemaphoreType.DMA,
        pltpu.SemaphoreType.DMA,
        pltpu.SemaphoreType.DMA,
        pltpu.SemaphoreType.DMA,
    ],
)
def _sc_loss(p1, p2, q1, q2, out, a_v, b_v, c_v, d_v, acc_v,
             s1, s2, s3, s4):
    wid = lax.axis_index("s") * 2 + lax.axis_index("c")
    col0 = wid * _CPT
    cp1 = pltpu.async_copy(p1.at[:, pl.ds(col0, _CPT)], a_v, s1)
    cp2 = pltpu.async_copy(q2.at[:, pl.ds(col0, _CPT)], b_v, s2)
    cp3 = pltpu.async_copy(p2.at[:, pl.ds(col0, _CPT)], c_v, s3)
    cp4 = pltpu.async_copy(q1.at[:, pl.ds(col0, _CPT)], d_v, s4)

    zero = jnp.zeros((16,), jnp.float32)
    nacc = 8

    def make_body(pr_v, po_v):
        def body(j, accs):
            accs = list(accs)
            c = j * 16
            for r in range(_K):
                x = pr_v[r, pl.ds(c, 16)]
                w = po_v[r, pl.ds(c, 16)]
                accs[r % nacc] = accs[r % nacc] + w * _log_term(x)
            return tuple(accs)
        return body

    cp1.wait()
    cp2.wait()
    accs = lax.fori_loop(0, _NT, make_body(a_v, b_v), (zero,) * nacc)
    acc1 = ((accs[0] + accs[1]) + (accs[2] + accs[3])) + \
           ((accs[4] + accs[5]) + (accs[6] + accs[7]))

    cp3.wait()
    cp4.wait()
    accs = lax.fori_loop(0, _NT, make_body(c_v, d_v), (zero,) * nacc)
    acc2 = ((accs[0] + accs[1]) + (accs[2] + accs[3])) + \
           ((accs[4] + accs[5]) + (accs[6] + accs[7]))

    acc_v[...] = acc1
    pltpu.sync_copy(acc_v, out.at[0, wid])
    acc_v[...] = acc2
    pltpu.sync_copy(acc_v, out.at[1, wid])


def _tc_body(p1, p2, q1, q2, out_ref):
    i = pl.program_id(0)

    @pl.when(i == 0)
    def _init():
        out_ref[...] = jnp.zeros_like(out_ref)

    def contrib(pr, po):
        x = jnp.clip(pr[...], 0.01, 0.99) + 1e-10
        return jnp.sum(po[...] * jnp.log(x), axis=1)

    out_ref[0, :] += contrib(p1, q2)
    out_ref[1, :] += contrib(p2, q1)


def _tc_loss(p1t, p2t, q1t, q2t):
    nblk = (_N - _SC_COLS) // _BC
    spec = pl.BlockSpec((_K, _BC), lambda i: (0, i + _SC_COLS // _BC))
    return pl.pallas_call(
        _tc_body, grid=(nblk,),
        in_specs=[spec, spec, spec, spec],
        out_specs=pl.BlockSpec((2, _K), lambda i: (0, 0)),
        out_shape=jax.ShapeDtypeStruct((2, _K), jnp.float32),
    )(p1t, p2t, q1t, q2t)


def kernel(prior_1, prior_2, post_1, post_2):
    t = (prior_1.T, prior_2.T, post_1.T, post_2.T)
    parts_sc = _sc_loss(*t)
    parts_tc = _tc_loss(*t)
    s = jnp.sum(parts_sc, axis=(1, 2)) + jnp.sum(parts_tc, axis=1)
    losses = 0.05 - s / _N
    return (losses[0], losses[1])


# speedup vs baseline: 2.1326x; 1.0736x over previous
import functools

import jax
import jax.numpy as jnp
from jax import lax
from jax.experimental import pallas as pl
from jax.experimental.pallas import tpu as pltpu
from jax.experimental.pallas import tpu_sc as plsc

_N, _K = 16384, 26
_NW = 32
_SC_COLS = 4096
_BC = 2048
_CPT = _SC_COLS // _NW
_NT = _CPT // 16

_LN2 = 0.6931471805599453
_C0 = 0.0001415121753789439 - 127.0 * _LN2
_C1 = 0.9954273382579881
_C2 = -0.4640725804471214
_C3 = 0.21641043832781495
_C4 = -0.05486285286206372


def _log_term(x):
    x = jnp.minimum(jnp.maximum(x, 0.01), 0.99)
    bits = lax.bitcast_convert_type(x, jnp.int32)
    eb = bits >> 23
    m = (bits & 0x7FFFFF) | 0x3F800000
    u = lax.bitcast_convert_type(m, jnp.float32) - 1.0
    r = _C4
    r = r * u + _C3
    r = r * u + _C2
    r = r * u + _C1
    r = r * u + _C0
    return eb.astype(jnp.float32) * _LN2 + r


_mesh = plsc.VectorSubcoreMesh(core_axis_name="c", subcore_axis_name="s")


@functools.partial(
    pl.kernel,
    mesh=_mesh,
    out_type=jax.ShapeDtypeStruct((2, _NW, 16), jnp.float32),
    scratch_types=[
        pltpu.VMEM((_K, _CPT), jnp.float32),
        pltpu.VMEM((_K, _CPT), jnp.float32),
        pltpu.VMEM((_K, _CPT), jnp.float32),
        pltpu.VMEM((_K, _CPT), jnp.float32),
        pltpu.VMEM((16,), jnp.float32),
        pltpu.SemaphoreType.DMA,
        pltpu.SemaphoreType.DMA,
        pltpu.SemaphoreType.DMA,
        pltpu.SemaphoreType.DMA,
    ],
)
def _sc_loss(p1, p2, q1, q2, out, a_v, b_v, c_v, d_v, acc_v,
             s1, s2, s3, s4):
    wid = lax.axis_index("s") * 2 + lax.axis_index("c")
    col0 = wid * _CPT
    cp1 = pltpu.async_copy(p1.at[:, pl.ds(col0, _CPT)], a_v, s1)
    cp2 = pltpu.async_copy(q2.at[:, pl.ds(col0, _CPT)], b_v, s2)
    cp3 = pltpu.async_copy(p2.at[:, pl.ds(col0, _CPT)], c_v, s3)
    cp4 = pltpu.async_copy(q1.at[:, pl.ds(col0, _CPT)], d_v, s4)

    zero = jnp.zeros((16,), jnp.float32)
    nacc = 8

    def make_body(pr_v, po_v):
        def body(j, accs):
            accs = list(accs)
            c = j * 16
            for r in range(_K):
                x = pr_v[r, pl.ds(c, 16)]
                w = po_v[r, pl.ds(c, 16)]
                accs[r % nacc] = accs[r % nacc] + w * _log_term(x)
            return tuple(accs)
        return body

    cp1.wait()
    cp2.wait()
    accs = lax.fori_loop(0, _NT, make_body(a_v, b_v), (zero,) * nacc)
    acc1 = ((accs[0] + accs[1]) + (accs[2] + accs[3])) + \
           ((accs[4] + accs[5]) + (accs[6] + accs[7]))

    cp3.wait()
    cp4.wait()
    accs = lax.fori_loop(0, _NT, make_body(c_v, d_v), (zero,) * nacc)
    acc2 = ((accs[0] + accs[1]) + (accs[2] + accs[3])) + \
           ((accs[4] + accs[5]) + (accs[6] + accs[7]))

    acc_v[...] = acc1
    pltpu.sync_copy(acc_v, out.at[0, wid])
    acc_v[...] = acc2
    pltpu.sync_copy(acc_v, out.at[1, wid])


def _tc_body(p1, p2, q1, q2, out_ref):
    i = pl.program_id(0)

    @pl.when(i == 0)
    def _init():
        out_ref[...] = jnp.zeros_like(out_ref)

    def contrib(pr, po):
        x = jnp.clip(pr[...], 0.01, 0.99) + 1e-10
        return jnp.sum(po[...] * jnp.log(x), axis=1)

    out_ref[0, :] += contrib(p1, q2)
    out_ref[1, :] += contrib(p2, q1)


def _tc_loss(p1t, p2t, q1t, q2t):
    nblk = (_N - _SC_COLS) // _BC
    spec = pl.BlockSpec((_K, _BC), lambda i: (0, i + _SC_COLS // _BC))
    return pl.pallas_call(
        _tc_body, grid=(nblk,),
        in_specs=[spec, spec, spec, spec],
        out_specs=pl.BlockSpec((2, _K), lambda i: (0, 0)),
        out_shape=jax.ShapeDtypeStruct((2, _K), jnp.float32),
    )(p1t, p2t, q1t, q2t)


def kernel(prior_1, prior_2, post_1, post_2):
    t = (prior_1.T, prior_2.T, post_1.T, post_2.T)
    parts_tc = _tc_loss(*t)
    parts_sc = _sc_loss(*t)
    s = jnp.sum(parts_sc, axis=(1, 2)) + jnp.sum(parts_tc, axis=1)
    losses = 0.05 - s / _N
    return (losses[0], losses[1])


# speedup vs baseline: 2.1684x; 1.0168x over previous
import functools

import jax
import jax.numpy as jnp
from jax import lax
from jax.experimental import pallas as pl
from jax.experimental.pallas import tpu as pltpu
from jax.experimental.pallas import tpu_sc as plsc

_N, _K = 16384, 26
_NW = 32
_SC_COLS = 4096
_BC = 2048
_CPT = _SC_COLS // _NW
_NT = _CPT // 16

_LN2 = 0.6931471805599453
_C0 = 0.0009250321113061233 - 127.0 * _LN2
_C1 = 0.9735508519008734
_C2 = -0.3921667221516742
_C3 = 0.11255014928628229


def _log_term(x):
    x = jnp.minimum(jnp.maximum(x, 0.01), 0.99)
    bits = lax.bitcast_convert_type(x, jnp.int32)
    eb = bits >> 23
    m = (bits & 0x7FFFFF) | 0x3F800000
    u = lax.bitcast_convert_type(m, jnp.float32) - 1.0
    r = _C3
    r = r * u + _C2
    r = r * u + _C1
    r = r * u + _C0
    return eb.astype(jnp.float32) * _LN2 + r


_mesh = plsc.VectorSubcoreMesh(core_axis_name="c", subcore_axis_name="s")


@functools.partial(
    pl.kernel,
    mesh=_mesh,
    out_type=jax.ShapeDtypeStruct((2, _NW, 16), jnp.float32),
    scratch_types=[
        pltpu.VMEM((_K, _CPT), jnp.float32),
        pltpu.VMEM((_K, _CPT), jnp.float32),
        pltpu.VMEM((_K, _CPT), jnp.float32),
        pltpu.VMEM((_K, _CPT), jnp.float32),
        pltpu.VMEM((16,), jnp.float32),
        pltpu.SemaphoreType.DMA,
        pltpu.SemaphoreType.DMA,
        pltpu.SemaphoreType.DMA,
        pltpu.SemaphoreType.DMA,
    ],
)
def _sc_loss(p1, p2, q1, q2, out, a_v, b_v, c_v, d_v, acc_v,
             s1, s2, s3, s4):
    wid = lax.axis_index("s") * 2 + lax.axis_index("c")
    col0 = wid * _CPT
    cp1 = pltpu.async_copy(p1.at[:, pl.ds(col0, _CPT)], a_v, s1)
    cp2 = pltpu.async_copy(q2.at[:, pl.ds(col0, _CPT)], b_v, s2)
    cp3 = pltpu.async_copy(p2.at[:, pl.ds(col0, _CPT)], c_v, s3)
    cp4 = pltpu.async_copy(q1.at[:, pl.ds(col0, _CPT)], d_v, s4)

    zero = jnp.zeros((16,), jnp.float32)
    nacc = 8

    def make_body(pr_v, po_v):
        def body(j, accs):
            accs = list(accs)
            c = j * 16
            for r in range(_K):
                x = pr_v[r, pl.ds(c, 16)]
                w = po_v[r, pl.ds(c, 16)]
                accs[r % nacc] = accs[r % nacc] + w * _log_term(x)
            return tuple(accs)
        return body

    cp1.wait()
    cp2.wait()
    accs = lax.fori_loop(0, _NT, make_body(a_v, b_v), (zero,) * nacc)
    acc1 = ((accs[0] + accs[1]) + (accs[2] + accs[3])) + \
           ((accs[4] + accs[5]) + (accs[6] + accs[7]))

    cp3.wait()
    cp4.wait()
    accs = lax.fori_loop(0, _NT, make_body(c_v, d_v), (zero,) * nacc)
    acc2 = ((accs[0] + accs[1]) + (accs[2] + accs[3])) + \
           ((accs[4] + accs[5]) + (accs[6] + accs[7]))

    acc_v[...] = acc1
    pltpu.sync_copy(acc_v, out.at[0, wid])
    acc_v[...] = acc2
    pltpu.sync_copy(acc_v, out.at[1, wid])


def _tc_body(p1, p2, q1, q2, out_ref):
    i = pl.program_id(0)

    @pl.when(i == 0)
    def _init():
        out_ref[...] = jnp.zeros_like(out_ref)

    def contrib(pr, po):
        x = jnp.clip(pr[...], 0.01, 0.99) + 1e-10
        return jnp.sum(po[...] * jnp.log(x), axis=1)

    out_ref[0, :] += contrib(p1, q2)
    out_ref[1, :] += contrib(p2, q1)


def _tc_loss(p1t, p2t, q1t, q2t):
    nblk = (_N - _SC_COLS) // _BC
    spec = pl.BlockSpec((_K, _BC), lambda i: (0, i + _SC_COLS // _BC))
    return pl.pallas_call(
        _tc_body, grid=(nblk,),
        in_specs=[spec, spec, spec, spec],
        out_specs=pl.BlockSpec((2, _K), lambda i: (0, 0)),
        out_shape=jax.ShapeDtypeStruct((2, _K), jnp.float32),
    )(p1t, p2t, q1t, q2t)


def kernel(prior_1, prior_2, post_1, post_2):
    t = (prior_1.T, prior_2.T, post_1.T, post_2.T)
    parts_tc = _tc_loss(*t)
    parts_sc = _sc_loss(*t)
    s = jnp.sum(parts_sc, axis=(1, 2)) + jnp.sum(parts_tc, axis=1)
    losses = 0.05 - s / _N
    return (losses[0], losses[1])
